# Initial kernel scaffold; baseline (speedup 1.0000x reference)
import jax
import jax.numpy as jnp
from jax.experimental import pallas as pl


def kernel(z, embedding):
    raise NotImplementedError("write your pallas kernel here")


# speedup vs baseline: 1.7753x; 1.7753x over previous
import jax
import jax.numpy as jnp
from jax.experimental import pallas as pl
from jax.experimental.pallas import tpu as pltpu

_NUM_CODES = 1024
_CODE_DIM = 64
_COMMITMENT_COST = 0.25
_BLOCK_TOKENS = 2048


def _vq_block_kernel(z_ref, emb_ref, zq_ref, idx_ref, loss_ref):
    z = z_ref[...]
    emb = emb_ref[...]
    z_sq = jnp.sum(z * z, axis=1, keepdims=True)
    e_sq = jnp.sum(emb * emb, axis=1)
    scores = jax.lax.dot_general(
        z, emb, (((1,), (1,)), ((), ())),
        preferred_element_type=jnp.float32)
    d = (z_sq - 2.0 * scores) + e_sq[None, :]
    idx = jnp.argmin(d, axis=1).astype(jnp.int32)
    dmin = jnp.min(d, axis=1)

    zq_ref[...] = z
    idx_ref[0, 0, :] = idx

    @pl.when(pl.program_id(0) == 0)
    def _init():
        loss_ref[0, 0] = 0.0

    loss_ref[0, 0] += jnp.sum(dmin)


def kernel(z, embedding):
    z_shape = z.shape
    z_flat = z.reshape(-1, _CODE_DIM)
    n_tokens = z_flat.shape[0]
    blk = _BLOCK_TOKENS
    grid = n_tokens // blk

    zq, idx3, loss_sum = pl.pallas_call(
        _vq_block_kernel,
        grid=(grid,),
        in_specs=[
            pl.BlockSpec((blk, _CODE_DIM), lambda i: (i, 0)),
            pl.BlockSpec((_NUM_CODES, _CODE_DIM), lambda i: (0, 0)),
        ],
        out_specs=[
            pl.BlockSpec((blk, _CODE_DIM), lambda i: (i, 0)),
            pl.BlockSpec((1, 1, blk), lambda i: (i, 0, 0)),
            pl.BlockSpec((1, 1), lambda i: (0, 0), memory_space=pltpu.SMEM),
        ],
        out_shape=[
            jax.ShapeDtypeStruct((n_tokens, _CODE_DIM), jnp.float32),
            jax.ShapeDtypeStruct((grid, 1, blk), jnp.int32),
            jax.ShapeDtypeStruct((1, 1), jnp.float32),
        ],
    )(z_flat, embedding)

    z_q_st = zq.reshape(z_shape)
    indices = idx3.reshape(z_shape[:-1])
    vq_loss = _COMMITMENT_COST * loss_sum[0, 0] / (n_tokens * _CODE_DIM)
    return (z_q_st, indices, vq_loss)


# speedup vs baseline: 1.9003x; 1.0704x over previous
import jax
import jax.numpy as jnp
from jax.experimental import pallas as pl
from jax.experimental.pallas import tpu as pltpu

_NUM_CODES = 1024
_CODE_DIM = 64
_COMMITMENT_COST = 0.25
_BLOCK_TOKENS = 2048


def _vq_block_kernel(z_ref, emb_ref, zq_ref, idx_ref, loss_ref):
    z = z_ref[...]
    emb = emb_ref[...]
    z_sq = jnp.sum(z * z, axis=1, keepdims=True)
    e_sq = jnp.sum(emb * emb, axis=1)
    scores = jax.lax.dot_general(
        z, emb, (((1,), (1,)), ((), ())),
        preferred_element_type=jnp.float32)
    d = (z_sq - 2.0 * scores) + e_sq[None, :]
    dmin = jnp.min(d, axis=1)
    iota = jax.lax.broadcasted_iota(jnp.int32, d.shape, 1)
    idx = jnp.min(
        jnp.where(d == dmin[:, None], iota, _NUM_CODES), axis=1
    ).astype(jnp.int32)

    zq_ref[...] = z
    idx_ref[0, 0, :] = idx

    @pl.when(pl.program_id(0) == 0)
    def _init():
        loss_ref[0, 0] = 0.0

    loss_ref[0, 0] += jnp.sum(dmin)


def kernel(z, embedding):
    z_shape = z.shape
    z_flat = z.reshape(-1, _CODE_DIM)
    n_tokens = z_flat.shape[0]
    blk = _BLOCK_TOKENS
    grid = n_tokens // blk

    zq, idx3, loss_sum = pl.pallas_call(
        _vq_block_kernel,
        grid=(grid,),
        in_specs=[
            pl.BlockSpec((blk, _CODE_DIM), lambda i: (i, 0)),
            pl.BlockSpec((_NUM_CODES, _CODE_DIM), lambda i: (0, 0)),
        ],
        out_specs=[
            pl.BlockSpec((blk, _CODE_DIM), lambda i: (i, 0)),
            pl.BlockSpec((1, 1, blk), lambda i: (i, 0, 0)),
            pl.BlockSpec((1, 1), lambda i: (0, 0), memory_space=pltpu.SMEM),
        ],
        out_shape=[
            jax.ShapeDtypeStruct((n_tokens, _CODE_DIM), jnp.float32),
            jax.ShapeDtypeStruct((grid, 1, blk), jnp.int32),
            jax.ShapeDtypeStruct((1, 1), jnp.float32),
        ],
    )(z_flat, embedding)

    z_q_st = zq.reshape(z_shape)
    indices = idx3.reshape(z_shape[:-1])
    vq_loss = _COMMITMENT_COST * loss_sum[0, 0] / (n_tokens * _CODE_DIM)
    return (z_q_st, indices, vq_loss)


# speedup vs baseline: 1.9593x; 1.0311x over previous
import jax
import jax.numpy as jnp
from jax.experimental import pallas as pl
from jax.experimental.pallas import tpu as pltpu

_NUM_CODES = 1024
_CODE_DIM = 64
_COMMITMENT_COST = 0.25
_BLOCK_TOKENS = 2048


def _vq_block_kernel(z_ref, emb_ref, zq_ref, idx_ref, loss_ref):
    z = z_ref[...]
    emb = emb_ref[...]
    z_sq = jnp.sum(z * z, axis=1, keepdims=True)
    e_sq = jnp.sum(emb * emb, axis=1)
    neg2_emb = -2.0 * emb
    scores2 = jax.lax.dot_general(
        z, neg2_emb, (((1,), (1,)), ((), ())),
        preferred_element_type=jnp.float32)
    d = (z_sq + scores2) + e_sq[None, :]
    dmin = jnp.min(d, axis=1)
    iota = jax.lax.broadcasted_iota(jnp.int32, d.shape, 1)
    idx = jnp.min(
        jnp.where(d == dmin[:, None], iota, _NUM_CODES), axis=1
    ).astype(jnp.int32)

    zq_ref[...] = z
    idx_ref[0, 0, :] = idx

    @pl.when(pl.program_id(0) == 0)
    def _init():
        loss_ref[0, 0] = 0.0

    loss_ref[0, 0] += jnp.sum(dmin)


def kernel(z, embedding):
    z_shape = z.shape
    z_flat = z.reshape(-1, _CODE_DIM)
    n_tokens = z_flat.shape[0]
    blk = _BLOCK_TOKENS
    grid = n_tokens // blk

    zq, idx3, loss_sum = pl.pallas_call(
        _vq_block_kernel,
        grid=(grid,),
        in_specs=[
            pl.BlockSpec((blk, _CODE_DIM), lambda i: (i, 0)),
            pl.BlockSpec((_NUM_CODES, _CODE_DIM), lambda i: (0, 0)),
        ],
        out_specs=[
            pl.BlockSpec((blk, _CODE_DIM), lambda i: (i, 0)),
            pl.BlockSpec((1, 1, blk), lambda i: (i, 0, 0)),
            pl.BlockSpec((1, 1), lambda i: (0, 0), memory_space=pltpu.SMEM),
        ],
        out_shape=[
            jax.ShapeDtypeStruct((n_tokens, _CODE_DIM), jnp.float32),
            jax.ShapeDtypeStruct((grid, 1, blk), jnp.int32),
            jax.ShapeDtypeStruct((1, 1), jnp.float32),
        ],
    )(z_flat, embedding)

    z_q_st = zq.reshape(z_shape)
    indices = idx3.reshape(z_shape[:-1])
    vq_loss = _COMMITMENT_COST * loss_sum[0, 0] / (n_tokens * _CODE_DIM)
    return (z_q_st, indices, vq_loss)


# speedup vs baseline: 1.9621x; 1.0015x over previous
import jax
import jax.numpy as jnp
from jax.experimental import pallas as pl
from jax.experimental.pallas import tpu as pltpu

_NUM_CODES = 1024
_CODE_DIM = 64
_COMMITMENT_COST = 0.25
_BLOCK_ROWS = 2


def _vq_block_kernel(z_ref, emb_ref, zq_ref, idx_ref, loss_ref):
    blk = _BLOCK_ROWS * 1024
    z = z_ref[...].reshape(blk, _CODE_DIM)
    emb = emb_ref[...]
    z_sq = jnp.sum(z * z, axis=1, keepdims=True)
    e_sq = jnp.sum(emb * emb, axis=1)
    neg2_emb = -2.0 * emb
    scores2 = jax.lax.dot_general(
        z, neg2_emb, (((1,), (1,)), ((), ())),
        preferred_element_type=jnp.float32)
    d = (z_sq + scores2) + e_sq[None, :]
    dmin = jnp.min(d, axis=1)
    iota = jax.lax.broadcasted_iota(jnp.int32, (1, _NUM_CODES), 1).astype(
        jnp.float32)
    idx = jnp.min(
        jnp.where(d == dmin[:, None], iota, float(_NUM_CODES)), axis=1
    ).astype(jnp.int32)

    zq_ref[...] = z_ref[...]
    idx_ref[0, :, :] = idx.reshape(_BLOCK_ROWS, 1024)

    @pl.when(pl.program_id(0) == 0)
    def _init():
        loss_ref[0, 0] = 0.0

    loss_ref[0, 0] += jnp.sum(dmin)


def kernel(z, embedding):
    rows = z.shape[0]
    grid = rows // _BLOCK_ROWS

    zq, idx3, loss_sum = pl.pallas_call(
        _vq_block_kernel,
        grid=(grid,),
        in_specs=[
            pl.BlockSpec((_BLOCK_ROWS, 1024, _CODE_DIM), lambda i: (i, 0, 0)),
            pl.BlockSpec((_NUM_CODES, _CODE_DIM), lambda i: (0, 0)),
        ],
        out_specs=[
            pl.BlockSpec((_BLOCK_ROWS, 1024, _CODE_DIM), lambda i: (i, 0, 0)),
            pl.BlockSpec((1, _BLOCK_ROWS, 1024), lambda i: (i, 0, 0)),
            pl.BlockSpec((1, 1), lambda i: (0, 0), memory_space=pltpu.SMEM),
        ],
        out_shape=[
            jax.ShapeDtypeStruct(z.shape, jnp.float32),
            jax.ShapeDtypeStruct((grid, _BLOCK_ROWS, 1024), jnp.int32),
            jax.ShapeDtypeStruct((1, 1), jnp.float32),
        ],
    )(z, embedding)

    indices = idx3.reshape(z.shape[:-1])
    vq_loss = _COMMITMENT_COST * loss_sum[0, 0] / (rows * 1024 * _CODE_DIM)
    return (zq, indices, vq_loss)
